# baseline (speedup 1.0000x reference)
import functools

import jax
import jax.numpy as jnp
from jax import lax
from jax.experimental import pallas as pl
from jax.experimental.pallas import tpu as pltpu
from jax.experimental.pallas import tpu_sc as plsc

D = 32
B = 4096
H = 50
BH = B * H
NC = 2
NS = 16
NW = NC * NS
NPW = BH // NW
C = 128
NCHUNK = NPW // C

_mesh = plsc.VectorSubcoreMesh(core_axis_name="c", subcore_axis_name="s")


@functools.partial(
    pl.kernel,
    out_type=(
        jax.ShapeDtypeStruct((BH, D), jnp.float32),
        jax.ShapeDtypeStruct((BH, D), jnp.float32),
    ),
    mesh=_mesh,
    compiler_params=pltpu.CompilerParams(use_tc_tiling_on_sc=False),
    scratch_types=[
        pltpu.VMEM((NCHUNK, C), jnp.int32),
        pltpu.VMEM((2, C, D), jnp.float32),
        pltpu.VMEM((2, C, D), jnp.float32),
        pltpu.SemaphoreType.DMA,
        pltpu.SemaphoreType.DMA,
        pltpu.SemaphoreType.DMA,
        pltpu.SemaphoreType.DMA,
    ],
)
def _gather2(x_hbm, real_hbm, imag_hbm, real_out, imag_out,
             idx_v, rbuf, ibuf, sr0, sr1, si0, si1):
    wid = lax.axis_index("s") * NC + lax.axis_index("c")
    base = wid * NPW
    pltpu.sync_copy(x_hbm.at[wid], idx_v)

    sems_r = (sr0, sr1)
    sems_i = (si0, si1)

    def start(g, b):
        pltpu.async_copy(real_hbm.at[idx_v.at[g]], rbuf.at[b], sems_r[b])
        pltpu.async_copy(imag_hbm.at[idx_v.at[g]], ibuf.at[b], sems_i[b])

    def finish(g, b):
        pltpu.make_async_copy(real_hbm.at[idx_v.at[g]], rbuf.at[b],
                              sems_r[b]).wait()
        pltpu.sync_copy(rbuf.at[b], real_out.at[pl.ds(base + g * C, C)])
        pltpu.make_async_copy(imag_hbm.at[idx_v.at[g]], ibuf.at[b],
                              sems_i[b]).wait()
        pltpu.sync_copy(ibuf.at[b], imag_out.at[pl.ds(base + g * C, C)])

    start(0, 0)

    def body(jj, carry):
        g0 = 2 * jj
        g1 = g0 + 1
        start(g1, 1)
        finish(g0, 0)

        @pl.when(g0 + 2 < NCHUNK)
        def _():
            start(g0 + 2, 0)

        finish(g1, 1)
        return carry

    lax.fori_loop(0, NCHUNK // 2, body, 0)


def kernel(x, real_table, imag_table):
    xw = x.reshape(NW, NCHUNK, C)
    r, i = _gather2(xw, real_table, imag_table)
    return lax.complex(r.reshape(B, H, D), i.reshape(B, H, D))


# speedup vs baseline: 1.0003x; 1.0003x over previous
import functools

import jax
import jax.numpy as jnp
from jax import lax
from jax.experimental import pallas as pl
from jax.experimental.pallas import tpu as pltpu
from jax.experimental.pallas import tpu_sc as plsc

D = 32
B = 4096
H = 50
BH = B * H
NC = 2
NS = 16
NW = NC * NS
NPW = BH // NW
C = 128
NCHUNK = NPW // C

_mesh = plsc.VectorSubcoreMesh(core_axis_name="c", subcore_axis_name="s")


@functools.partial(
    pl.kernel,
    out_type=(
        jax.ShapeDtypeStruct((BH, D), jnp.float32),
        jax.ShapeDtypeStruct((BH, D), jnp.float32),
    ),
    mesh=_mesh,
    compiler_params=pltpu.CompilerParams(use_tc_tiling_on_sc=False),
    scratch_types=[
        pltpu.VMEM((NCHUNK, C), jnp.int32),
        pltpu.VMEM((2, C, D), jnp.float32),
        pltpu.VMEM((2, C, D), jnp.float32),
        pltpu.SemaphoreType.DMA,
        pltpu.SemaphoreType.DMA,
        pltpu.SemaphoreType.DMA,
        pltpu.SemaphoreType.DMA,
    ],
)
def _gather2(x_hbm, real_hbm, imag_hbm, real_out, imag_out,
             idx_v, rbuf, ibuf, sr0, sr1, si0, si1):
    wid = lax.axis_index("s") * NC + lax.axis_index("c")
    base = wid * NPW
    pltpu.sync_copy(x_hbm.at[wid], idx_v)

    sems_r = (sr0, sr1)
    sems_i = (si0, si1)

    def start(g, b):
        idx = idx_v.at[g]
        pltpu.async_copy(real_hbm.at[idx], rbuf.at[b], sems_r[b])
        pltpu.async_copy(imag_hbm.at[idx], ibuf.at[b], sems_i[b])

    def finish(g, b):
        idx = idx_v.at[g]
        pltpu.make_async_copy(real_hbm.at[idx], rbuf.at[b], sems_r[b]).wait()
        pltpu.sync_copy(rbuf.at[b], real_out.at[pl.ds(base + g * C, C)])
        pltpu.make_async_copy(imag_hbm.at[idx], ibuf.at[b], sems_i[b]).wait()
        pltpu.sync_copy(ibuf.at[b], imag_out.at[pl.ds(base + g * C, C)])

    start(0, 0)

    def body(jj, carry):
        g0 = 2 * jj
        g1 = g0 + 1
        start(g1, 1)
        finish(g0, 0)

        @pl.when(g0 + 2 < NCHUNK)
        def _():
            start(g0 + 2, 0)

        finish(g1, 1)
        return carry

    lax.fori_loop(0, NCHUNK // 2, body, 0)


def kernel(x, real_table, imag_table):
    xw = x.reshape(NW, NCHUNK, C)
    r, i = _gather2(xw, real_table, imag_table)
    z = lax.complex(r, i)
    return z.reshape(B, H, D)
